# Initial kernel scaffold; baseline (speedup 1.0000x reference)
import jax
import jax.numpy as jnp
from jax.experimental import pallas as pl


def kernel(x, edge_index, edge_weight, W1, b1, W2, b2):
    raise NotImplementedError("write your pallas kernel here")


# speedup vs baseline: 39.6897x; 39.6897x over previous
import functools

import jax
import jax.numpy as jnp
from jax import lax
from jax.experimental import pallas as pl
from jax.experimental.pallas import tpu as pltpu
from jax.experimental.pallas import tpu_sc as plsc

NC = 2
NS = 16
NW = NC * NS
CH = 1024
LANE = 128


def _wid():
    return lax.axis_index("c") * NS + lax.axis_index("s")


def _stage_slices(n):
    per = n // NS
    return per


def _deg_body(nchunks, npad, col_hbm, zeros_hbm, deg_out, col_v, ones_v, deg_sh):
    c = lax.axis_index("c")
    s = lax.axis_index("s")
    wid = c * NS + s
    per = npad // NS
    pltpu.sync_copy(zeros_hbm.at[pl.ds(s * per, per)], deg_sh.at[pl.ds(s * per, per)])
    for k in range(8):
        ones_v[pl.ds(k * 16, 16)] = jnp.full((16,), 1.0, jnp.float32)
    plsc.subcore_barrier()

    rows_per_worker = nchunks * (CH // LANE)

    def chunk(ci, carry):
        rb = wid * rows_per_worker + ci * (CH // LANE)
        pltpu.sync_copy(col_hbm.at[pl.ds(rb, CH // LANE)], col_v)
        for j in range(CH // LANE):
            pltpu.sync_copy(ones_v, deg_sh.at[col_v.at[j]], add=True)
        return carry

    lax.fori_loop(0, nchunks, chunk, 0)
    plsc.subcore_barrier()
    pltpu.sync_copy(deg_sh.at[pl.ds(s * per, per)],
                    deg_out.at[pl.ds(c * npad + s * per, per)])


def _agg_body(nchunks, npad, width, row_hbm, col_hbm, ew_hbm, tab_hbm, zeros_hbm,
              acc_out, row_v, col_v, ew_v, ridx, cidx, gt_v, tab_sh, acc_sh):
    c = lax.axis_index("c")
    s = lax.axis_index("s")
    wid = c * NS + s
    per = (npad // NS) * width
    sl = pl.ds(s * per, per)
    pltpu.sync_copy(tab_hbm.at[sl], tab_sh.at[sl])
    pltpu.sync_copy(zeros_hbm.at[sl], acc_sh.at[sl])
    plsc.subcore_barrier()

    rows_per_worker = nchunks * (CH // LANE)

    def chunk(ci, carry):
        rb = wid * rows_per_worker + ci * (CH // LANE)
        pltpu.sync_copy(row_hbm.at[pl.ds(rb, CH // LANE)], row_v)
        pltpu.sync_copy(col_hbm.at[pl.ds(rb, CH // LANE)], col_v)
        pltpu.sync_copy(ew_hbm.at[pl.ds(rb * LANE, CH)], ew_v)
        for j in range(CH // LANE):

            def idx_body(m, carry2):
                off = pl.ds(m * 16, 16)
                rv = row_v[j, off] * width
                cv = col_v[j, off] * width
                for k in range(width):
                    ridx[k, off] = rv + k
                    cidx[k, off] = cv + k
                return carry2

            lax.fori_loop(0, LANE // 16, idx_body, 0)
            for k in range(width):
                pltpu.sync_copy(tab_sh.at[ridx.at[k]], gt_v.at[k])

            def mul_body(m, carry2):
                off = pl.ds(m * 16, 16)
                ewv = ew_v[pl.ds(j * LANE + m * 16, 16)]
                for k in range(width):
                    gt_v[k, off] = gt_v[k, off] * ewv
                return carry2

            lax.fori_loop(0, LANE // 16, mul_body, 0)
            for k in range(width):
                pltpu.sync_copy(gt_v.at[k], acc_sh.at[cidx.at[k]], add=True)
        return carry

    lax.fori_loop(0, nchunks, chunk, 0)
    plsc.subcore_barrier()
    pltpu.sync_copy(acc_sh.at[sl], acc_out.at[pl.ds(c * NS * per + s * per, per)])


def _make_sc_kernels(npad, nchunks):
    mesh = plsc.VectorSubcoreMesh(core_axis_name="c", subcore_axis_name="s",
                                  num_cores=NC, num_subcores=NS)
    params = pltpu.CompilerParams(needs_layout_passes=False,
                                  use_tc_tiling_on_sc=False)
    deg_k = pl.kernel(
        functools.partial(_deg_body, nchunks, npad),
        out_type=jax.ShapeDtypeStruct((NC * npad,), jnp.float32),
        mesh=mesh,
        compiler_params=params,
        scratch_types=[
            pltpu.VMEM((CH // LANE, LANE), jnp.int32),
            pltpu.VMEM((LANE,), jnp.float32),
            pltpu.VMEM_SHARED((npad,), jnp.float32),
        ],
    )

    def agg_k(width):
        return pl.kernel(
            functools.partial(_agg_body, nchunks, npad, width),
            out_type=jax.ShapeDtypeStruct((NC * npad * width,), jnp.float32),
            mesh=mesh,
            compiler_params=params,
            scratch_types=[
                pltpu.VMEM((CH // LANE, LANE), jnp.int32),
                pltpu.VMEM((CH // LANE, LANE), jnp.int32),
                pltpu.VMEM((CH,), jnp.float32),
                pltpu.VMEM((width, LANE), jnp.int32),
                pltpu.VMEM((width, LANE), jnp.int32),
                pltpu.VMEM((width, LANE), jnp.float32),
                pltpu.VMEM_SHARED((npad * width,), jnp.float32),
                pltpu.VMEM_SHARED((npad * width,), jnp.float32),
            ],
        )

    return deg_k, agg_k(4), agg_k(2)


def _tc_prep(degp_ref, xt_ref, xw_ref, dinv_ref, inv_ref):
    deg = degp_ref[0] + degp_ref[1] + 1.0
    dinv = lax.rsqrt(deg)
    inv = 1.0 / deg
    dinv_ref[...] = dinv
    inv_ref[...] = inv
    for j in range(3):
        xw_ref[j] = dinv * xt_ref[j]
    xw_ref[3] = dinv


def _tc_mid(s0_ref, s1_ref, xt_ref, dinv_ref, inv_ref, w1_ref, b1_ref,
            w2_ref, b2_ref, u_ref, g_ref, t_ref):
    dinv = dinv_ref[...]
    inv = inv_ref[...]
    sc = [s0_ref[j] + s1_ref[j] for j in range(4)]
    t_ref[...] = sc[3]
    g = [None, None]
    for cc in range(2):
        acc = None
        for k in range(16):
            hk = (sc[0] * w1_ref[k, 0] + sc[1] * w1_ref[k, 1]
                  + sc[2] * w1_ref[k, 2] + sc[3] * b1_ref[k]) * dinv
            hk = hk + (xt_ref[0] * w1_ref[k, 0] + xt_ref[1] * w1_ref[k, 1]
                       + xt_ref[2] * w1_ref[k, 2] + b1_ref[k]) * inv
            rk = jnp.maximum(hk, 0.0)
            term = rk * w2_ref[cc, k]
            acc = term if acc is None else acc + term
        g[cc] = acc
    for cc in range(2):
        g_ref[cc] = g[cc]
        u_ref[cc] = dinv * g[cc]


def _tc_fin(v0_ref, v1_ref, t_ref, dinv_ref, inv_ref, g_ref, b2_ref, o_ref):
    dinv = dinv_ref[...]
    inv = inv_ref[...]
    t = t_ref[...]
    h = [None, None]
    for cc in range(2):
        v = v0_ref[cc] + v1_ref[cc]
        h[cc] = dinv * (v + t * b2_ref[cc]) + inv * (g_ref[cc] + b2_ref[cc])
    m = jnp.maximum(h[0], h[1])
    lse = m + jnp.log(jnp.exp(h[0] - m) + jnp.exp(h[1] - m))
    for cc in range(2):
        o_ref[cc] = h[cc] - lse


def kernel(x, edge_index, edge_weight, W1, b1, W2, b2):
    n = x.shape[0]
    ne = edge_index.shape[1]
    npad = ((n + NW * LANE - 1) // (NW * LANE)) * (NW * LANE)
    nrows = npad // LANE
    nchunks = (ne + NW * CH - 1) // (NW * CH)
    nepad = NW * nchunks * CH
    pad = nepad - ne

    row = edge_index[0].astype(jnp.int32)
    col = edge_index[1].astype(jnp.int32)
    ew = edge_weight.astype(jnp.float32)
    if pad:
        spread = n + (jnp.arange(pad, dtype=jnp.int32) % (npad - n))
        row = jnp.concatenate([row, spread])
        col = jnp.concatenate([col, spread])
        ew = jnp.concatenate([ew, jnp.zeros((pad,), jnp.float32)])
    row2d = row.reshape(nepad // LANE, LANE)
    col2d = col.reshape(nepad // LANE, LANE)

    xp = jnp.pad(x.astype(jnp.float32), ((0, npad - n), (0, 0)))
    xt = xp.T.reshape(3, nrows, LANE)
    zeros1 = jnp.zeros((npad,), jnp.float32)
    zeros4 = jnp.zeros((npad * 4,), jnp.float32)
    zeros2 = jnp.zeros((npad * 2,), jnp.float32)

    deg_k, agg4_k, agg2_k = _make_sc_kernels(npad, nchunks)

    degp = deg_k(col2d, zeros1).reshape(NC, nrows, LANE)

    smem = pl.BlockSpec(memory_space=pltpu.SMEM)
    xw_t, dinv, inv = pl.pallas_call(
        _tc_prep,
        out_shape=[jax.ShapeDtypeStruct((4, nrows, LANE), jnp.float32),
                   jax.ShapeDtypeStruct((nrows, LANE), jnp.float32),
                   jax.ShapeDtypeStruct((nrows, LANE), jnp.float32)],
    )(degp, xt)
    xw = xw_t.reshape(4, npad).T.reshape(npad * 4)

    sp = agg4_k(row2d, col2d, ew, xw, zeros4)
    sp_t = sp.reshape(NC, npad, 4).transpose(0, 2, 1).reshape(NC, 4, nrows, LANE)

    u_t, g_t, tcol = pl.pallas_call(
        _tc_mid,
        in_specs=[pl.BlockSpec(), pl.BlockSpec(), pl.BlockSpec(),
                  pl.BlockSpec(), pl.BlockSpec(), smem, smem, smem, smem],
        out_shape=[jax.ShapeDtypeStruct((2, nrows, LANE), jnp.float32),
                   jax.ShapeDtypeStruct((2, nrows, LANE), jnp.float32),
                   jax.ShapeDtypeStruct((nrows, LANE), jnp.float32)],
    )(sp_t[0], sp_t[1], xt, dinv, inv, W1, b1, W2, b2)
    u = u_t.reshape(2, npad).T.reshape(npad * 2)

    vp = agg2_k(row2d, col2d, ew, u, zeros2)
    vp_t = vp.reshape(NC, npad, 2).transpose(0, 2, 1).reshape(NC, 2, nrows, LANE)

    o_t = pl.pallas_call(
        _tc_fin,
        in_specs=[pl.BlockSpec(), pl.BlockSpec(), pl.BlockSpec(),
                  pl.BlockSpec(), pl.BlockSpec(), pl.BlockSpec(), smem],
        out_shape=jax.ShapeDtypeStruct((2, nrows, LANE), jnp.float32),
    )(vp_t[0], vp_t[1], tcol, dinv, inv, g_t, b2)
    return o_t.reshape(2, npad).T[:n]


# speedup vs baseline: 74.6587x; 1.8811x over previous
import functools

import jax
import jax.numpy as jnp
from jax import lax
from jax.experimental import pallas as pl
from jax.experimental.pallas import tpu as pltpu
from jax.experimental.pallas import tpu_sc as plsc

NC = 2
NS = 16
NW = NC * NS
CH = 1024
LANE = 128


def _wid():
    return lax.axis_index("c") * NS + lax.axis_index("s")


def _stage_slices(n):
    per = n // NS
    return per


def _deg_body(nchunks, npad, col_hbm, zeros_hbm, deg_out, col_v, ones_v, deg_sh,
              sem):
    c = lax.axis_index("c")
    s = lax.axis_index("s")
    wid = c * NS + s
    per = npad // NS
    pltpu.sync_copy(zeros_hbm.at[pl.ds(s * per, per)], deg_sh.at[pl.ds(s * per, per)])
    for k in range(8):
        ones_v[pl.ds(k * 16, 16)] = jnp.full((16,), 1.0, jnp.float32)
    plsc.subcore_barrier()

    rows_per_worker = nchunks * (CH // LANE)

    def chunk(ci, carry):
        rb = wid * rows_per_worker + ci * (CH // LANE)
        pltpu.sync_copy(col_hbm.at[pl.ds(rb, CH // LANE)], col_v)
        descs = [pltpu.async_copy(ones_v, deg_sh.at[col_v.at[j]], sem, add=True)
                 for j in range(CH // LANE)]
        for d in descs:
            d.wait()
        return carry

    lax.fori_loop(0, nchunks, chunk, 0)
    plsc.subcore_barrier()
    pltpu.sync_copy(deg_sh.at[pl.ds(s * per, per)],
                    deg_out.at[pl.ds(c * npad + s * per, per)])


def _agg_body(nchunks, npad, width, row_hbm, col_hbm, ew_hbm, tab_hbm, zeros_hbm,
              acc_out, row_v, col_v, ew_v, ridx, cidx, gt_v, tab_sh, acc_sh,
              semg, sems):
    c = lax.axis_index("c")
    s = lax.axis_index("s")
    wid = c * NS + s
    per = (npad // NS) * width
    sl = pl.ds(s * per, per)
    pltpu.sync_copy(tab_hbm.at[sl], tab_sh.at[sl])
    pltpu.sync_copy(zeros_hbm.at[sl], acc_sh.at[sl])
    plsc.subcore_barrier()

    rows_per_worker = nchunks * (CH // LANE)

    nsub = CH // LANE

    def chunk(ci, carry):
        rb = wid * rows_per_worker + ci * nsub
        lr = pltpu.async_copy(row_hbm.at[pl.ds(rb, nsub)], row_v, semg)
        lc = pltpu.async_copy(col_hbm.at[pl.ds(rb, nsub)], col_v, semg)
        le = pltpu.async_copy(ew_hbm.at[pl.ds(rb * LANE, CH)], ew_v, semg)
        lr.wait(); lc.wait(); le.wait()

        for j in range(nsub):

            def idx_body(m, carry2, j=j):
                off = pl.ds(m * 16, 16)
                rv = row_v[j, off] * width
                cv = col_v[j, off] * width
                for k in range(width):
                    ridx[j * width + k, off] = rv + k
                    cidx[j * width + k, off] = cv + k
                return carry2

            lax.fori_loop(0, LANE // 16, idx_body, 0)

        gd = [pltpu.async_copy(tab_sh.at[ridx.at[t]], gt_v.at[t], semg)
              for t in range(nsub * width)]
        for d in gd:
            d.wait()

        for j in range(nsub):

            def mul_body(m, carry2, j=j):
                off = pl.ds(m * 16, 16)
                ewv = ew_v[pl.ds(j * LANE + m * 16, 16)]
                for k in range(width):
                    gt_v[j * width + k, off] = gt_v[j * width + k, off] * ewv
                return carry2

            lax.fori_loop(0, LANE // 16, mul_body, 0)

        sd = [pltpu.async_copy(gt_v.at[t], acc_sh.at[cidx.at[t]], sems, add=True)
              for t in range(nsub * width)]
        for d in sd:
            d.wait()
        return carry

    lax.fori_loop(0, nchunks, chunk, 0)
    plsc.subcore_barrier()
    pltpu.sync_copy(acc_sh.at[sl], acc_out.at[pl.ds(c * NS * per + s * per, per)])


def _make_sc_kernels(npad, nchunks):
    mesh = plsc.VectorSubcoreMesh(core_axis_name="c", subcore_axis_name="s",
                                  num_cores=NC, num_subcores=NS)
    params = pltpu.CompilerParams(needs_layout_passes=False,
                                  use_tc_tiling_on_sc=False)
    deg_k = pl.kernel(
        functools.partial(_deg_body, nchunks, npad),
        out_type=jax.ShapeDtypeStruct((NC * npad,), jnp.float32),
        mesh=mesh,
        compiler_params=params,
        scratch_types=[
            pltpu.VMEM((CH // LANE, LANE), jnp.int32),
            pltpu.VMEM((LANE,), jnp.float32),
            pltpu.VMEM_SHARED((npad,), jnp.float32),
            pltpu.SemaphoreType.DMA,
        ],
    )

    def agg_k(width):
        return pl.kernel(
            functools.partial(_agg_body, nchunks, npad, width),
            out_type=jax.ShapeDtypeStruct((NC * npad * width,), jnp.float32),
            mesh=mesh,
            compiler_params=params,
            scratch_types=[
                pltpu.VMEM((CH // LANE, LANE), jnp.int32),
                pltpu.VMEM((CH // LANE, LANE), jnp.int32),
                pltpu.VMEM((CH,), jnp.float32),
                pltpu.VMEM((CH // LANE * width, LANE), jnp.int32),
                pltpu.VMEM((CH // LANE * width, LANE), jnp.int32),
                pltpu.VMEM((CH // LANE * width, LANE), jnp.float32),
                pltpu.VMEM_SHARED((npad * width,), jnp.float32),
                pltpu.VMEM_SHARED((npad * width,), jnp.float32),
                pltpu.SemaphoreType.DMA,
                pltpu.SemaphoreType.DMA,
            ],
        )

    return deg_k, agg_k(4), agg_k(2)


def _tc_prep(degp_ref, xt_ref, xw_ref, dinv_ref, inv_ref):
    deg = degp_ref[0] + degp_ref[1] + 1.0
    dinv = lax.rsqrt(deg)
    inv = 1.0 / deg
    dinv_ref[...] = dinv
    inv_ref[...] = inv
    for j in range(3):
        xw_ref[j] = dinv * xt_ref[j]
    xw_ref[3] = dinv


def _tc_mid(s0_ref, s1_ref, xt_ref, dinv_ref, inv_ref, w1_ref, b1_ref,
            w2_ref, b2_ref, u_ref, g_ref, t_ref):
    dinv = dinv_ref[...]
    inv = inv_ref[...]
    sc = [s0_ref[j] + s1_ref[j] for j in range(4)]
    t_ref[...] = sc[3]
    g = [None, None]
    for cc in range(2):
        acc = None
        for k in range(16):
            hk = (sc[0] * w1_ref[k, 0] + sc[1] * w1_ref[k, 1]
                  + sc[2] * w1_ref[k, 2] + sc[3] * b1_ref[k]) * dinv
            hk = hk + (xt_ref[0] * w1_ref[k, 0] + xt_ref[1] * w1_ref[k, 1]
                       + xt_ref[2] * w1_ref[k, 2] + b1_ref[k]) * inv
            rk = jnp.maximum(hk, 0.0)
            term = rk * w2_ref[cc, k]
            acc = term if acc is None else acc + term
        g[cc] = acc
    for cc in range(2):
        g_ref[cc] = g[cc]
        u_ref[cc] = dinv * g[cc]


def _tc_fin(v0_ref, v1_ref, t_ref, dinv_ref, inv_ref, g_ref, b2_ref, o_ref):
    dinv = dinv_ref[...]
    inv = inv_ref[...]
    t = t_ref[...]
    h = [None, None]
    for cc in range(2):
        v = v0_ref[cc] + v1_ref[cc]
        h[cc] = dinv * (v + t * b2_ref[cc]) + inv * (g_ref[cc] + b2_ref[cc])
    m = jnp.maximum(h[0], h[1])
    lse = m + jnp.log(jnp.exp(h[0] - m) + jnp.exp(h[1] - m))
    for cc in range(2):
        o_ref[cc] = h[cc] - lse


def kernel(x, edge_index, edge_weight, W1, b1, W2, b2):
    n = x.shape[0]
    ne = edge_index.shape[1]
    npad = ((n + NW * LANE - 1) // (NW * LANE)) * (NW * LANE)
    nrows = npad // LANE
    nchunks = (ne + NW * CH - 1) // (NW * CH)
    nepad = NW * nchunks * CH
    pad = nepad - ne

    row = edge_index[0].astype(jnp.int32)
    col = edge_index[1].astype(jnp.int32)
    ew = edge_weight.astype(jnp.float32)
    if pad:
        spread = n + (jnp.arange(pad, dtype=jnp.int32) % (npad - n))
        row = jnp.concatenate([row, spread])
        col = jnp.concatenate([col, spread])
        ew = jnp.concatenate([ew, jnp.zeros((pad,), jnp.float32)])
    row2d = row.reshape(nepad // LANE, LANE)
    col2d = col.reshape(nepad // LANE, LANE)

    xp = jnp.pad(x.astype(jnp.float32), ((0, npad - n), (0, 0)))
    xt = xp.T.reshape(3, nrows, LANE)
    zeros1 = jnp.zeros((npad,), jnp.float32)
    zeros4 = jnp.zeros((npad * 4,), jnp.float32)
    zeros2 = jnp.zeros((npad * 2,), jnp.float32)

    deg_k, agg4_k, agg2_k = _make_sc_kernels(npad, nchunks)

    degp = deg_k(col2d, zeros1).reshape(NC, nrows, LANE)

    smem = pl.BlockSpec(memory_space=pltpu.SMEM)
    xw_t, dinv, inv = pl.pallas_call(
        _tc_prep,
        out_shape=[jax.ShapeDtypeStruct((4, nrows, LANE), jnp.float32),
                   jax.ShapeDtypeStruct((nrows, LANE), jnp.float32),
                   jax.ShapeDtypeStruct((nrows, LANE), jnp.float32)],
    )(degp, xt)
    xw = xw_t.reshape(4, npad).T.reshape(npad * 4)

    sp = agg4_k(row2d, col2d, ew, xw, zeros4)
    sp_t = sp.reshape(NC, npad, 4).transpose(0, 2, 1).reshape(NC, 4, nrows, LANE)

    u_t, g_t, tcol = pl.pallas_call(
        _tc_mid,
        in_specs=[pl.BlockSpec(), pl.BlockSpec(), pl.BlockSpec(),
                  pl.BlockSpec(), pl.BlockSpec(), smem, smem, smem, smem],
        out_shape=[jax.ShapeDtypeStruct((2, nrows, LANE), jnp.float32),
                   jax.ShapeDtypeStruct((2, nrows, LANE), jnp.float32),
                   jax.ShapeDtypeStruct((nrows, LANE), jnp.float32)],
    )(sp_t[0], sp_t[1], xt, dinv, inv, W1, b1, W2, b2)
    u = u_t.reshape(2, npad).T.reshape(npad * 2)

    vp = agg2_k(row2d, col2d, ew, u, zeros2)
    vp_t = vp.reshape(NC, npad, 2).transpose(0, 2, 1).reshape(NC, 2, nrows, LANE)

    o_t = pl.pallas_call(
        _tc_fin,
        in_specs=[pl.BlockSpec(), pl.BlockSpec(), pl.BlockSpec(),
                  pl.BlockSpec(), pl.BlockSpec(), pl.BlockSpec(), smem],
        out_shape=jax.ShapeDtypeStruct((2, nrows, LANE), jnp.float32),
    )(vp_t[0], vp_t[1], tcol, dinv, inv, g_t, b2)
    return o_t.reshape(2, npad).T[:n]


# speedup vs baseline: 80.6057x; 1.0797x over previous
import functools

import jax
import jax.numpy as jnp
from jax import lax
from jax.experimental import pallas as pl
from jax.experimental.pallas import tpu as pltpu
from jax.experimental.pallas import tpu_sc as plsc

NC = 2
NS = 16
NW = NC * NS
CH = 2048
LANE = 128


def _wid():
    return lax.axis_index("c") * NS + lax.axis_index("s")


def _stage_slices(n):
    per = n // NS
    return per


def _deg_body(nchunks, npad, col_hbm, zeros_hbm, deg_out, col_v, ones_v, deg_sh,
              sem):
    c = lax.axis_index("c")
    s = lax.axis_index("s")
    wid = c * NS + s
    per = npad // NS
    pltpu.sync_copy(zeros_hbm.at[pl.ds(s * per, per)], deg_sh.at[pl.ds(s * per, per)])
    for k in range(8):
        ones_v[pl.ds(k * 16, 16)] = jnp.full((16,), 1.0, jnp.float32)
    plsc.subcore_barrier()

    rows_per_worker = nchunks * (CH // LANE)

    def chunk(ci, carry):
        rb = wid * rows_per_worker + ci * (CH // LANE)
        pltpu.sync_copy(col_hbm.at[pl.ds(rb, CH // LANE)], col_v)
        descs = [pltpu.async_copy(ones_v, deg_sh.at[col_v.at[j]], sem, add=True)
                 for j in range(CH // LANE)]
        for d in descs:
            d.wait()
        return carry

    lax.fori_loop(0, nchunks, chunk, 0)
    plsc.subcore_barrier()
    pltpu.sync_copy(deg_sh.at[pl.ds(s * per, per)],
                    deg_out.at[pl.ds(c * npad + s * per, per)])


def _agg_body(nchunks, npad, width, row_hbm, col_hbm, ew_hbm, tab_hbm, zeros_hbm,
              acc_out, row_v, col_v, ew_v, ridx, cidx, gt_v, tab_sh, acc_sh,
              semg, sems):
    c = lax.axis_index("c")
    s = lax.axis_index("s")
    wid = c * NS + s
    per = (npad // NS) * width
    sl = pl.ds(s * per, per)
    pltpu.sync_copy(tab_hbm.at[sl], tab_sh.at[sl])
    pltpu.sync_copy(zeros_hbm.at[sl], acc_sh.at[sl])
    plsc.subcore_barrier()

    rows_per_worker = nchunks * (CH // LANE)

    nsub = CH // LANE

    def chunk(ci, carry):
        rb = wid * rows_per_worker + ci * nsub
        lr = pltpu.async_copy(row_hbm.at[pl.ds(rb, nsub)], row_v, semg)
        lc = pltpu.async_copy(col_hbm.at[pl.ds(rb, nsub)], col_v, semg)
        le = pltpu.async_copy(ew_hbm.at[pl.ds(rb * LANE, CH)], ew_v, semg)
        lr.wait(); lc.wait(); le.wait()

        for j in range(nsub):

            def idx_body(m, carry2, j=j):
                off = pl.ds(m * 16, 16)
                rv = row_v[j, off] * width
                cv = col_v[j, off] * width
                for k in range(width):
                    ridx[j * width + k, off] = rv + k
                    cidx[j * width + k, off] = cv + k
                return carry2

            lax.fori_loop(0, LANE // 16, idx_body, 0)

        gd = [pltpu.async_copy(tab_sh.at[ridx.at[t]], gt_v.at[t], semg)
              for t in range(nsub * width)]
        for d in gd:
            d.wait()

        for j in range(nsub):

            def mul_body(m, carry2, j=j):
                off = pl.ds(m * 16, 16)
                ewv = ew_v[pl.ds(j * LANE + m * 16, 16)]
                for k in range(width):
                    gt_v[j * width + k, off] = gt_v[j * width + k, off] * ewv
                return carry2

            lax.fori_loop(0, LANE // 16, mul_body, 0)

        sd = [pltpu.async_copy(gt_v.at[t], acc_sh.at[cidx.at[t]], sems, add=True)
              for t in range(nsub * width)]
        for d in sd:
            d.wait()
        return carry

    lax.fori_loop(0, nchunks, chunk, 0)
    plsc.subcore_barrier()
    pltpu.sync_copy(acc_sh.at[sl], acc_out.at[pl.ds(c * NS * per + s * per, per)])


def _make_sc_kernels(npad, nchunks):
    mesh = plsc.VectorSubcoreMesh(core_axis_name="c", subcore_axis_name="s",
                                  num_cores=NC, num_subcores=NS)
    params = pltpu.CompilerParams(needs_layout_passes=False,
                                  use_tc_tiling_on_sc=False)
    deg_k = pl.kernel(
        functools.partial(_deg_body, nchunks, npad),
        out_type=jax.ShapeDtypeStruct((NC * npad,), jnp.float32),
        mesh=mesh,
        compiler_params=params,
        scratch_types=[
            pltpu.VMEM((CH // LANE, LANE), jnp.int32),
            pltpu.VMEM((LANE,), jnp.float32),
            pltpu.VMEM_SHARED((npad,), jnp.float32),
            pltpu.SemaphoreType.DMA,
        ],
    )

    def agg_k(width):
        return pl.kernel(
            functools.partial(_agg_body, nchunks, npad, width),
            out_type=jax.ShapeDtypeStruct((NC * npad * width,), jnp.float32),
            mesh=mesh,
            compiler_params=params,
            scratch_types=[
                pltpu.VMEM((CH // LANE, LANE), jnp.int32),
                pltpu.VMEM((CH // LANE, LANE), jnp.int32),
                pltpu.VMEM((CH,), jnp.float32),
                pltpu.VMEM((CH // LANE * width, LANE), jnp.int32),
                pltpu.VMEM((CH // LANE * width, LANE), jnp.int32),
                pltpu.VMEM((CH // LANE * width, LANE), jnp.float32),
                pltpu.VMEM_SHARED((npad * width,), jnp.float32),
                pltpu.VMEM_SHARED((npad * width,), jnp.float32),
                pltpu.SemaphoreType.DMA,
                pltpu.SemaphoreType.DMA,
            ],
        )

    return deg_k, agg_k(4), agg_k(2)


def _tc_prep(degp_ref, xt_ref, xw_ref, dinv_ref, inv_ref):
    deg = degp_ref[0] + degp_ref[1] + 1.0
    dinv = lax.rsqrt(deg)
    inv = 1.0 / deg
    dinv_ref[...] = dinv
    inv_ref[...] = inv
    for j in range(3):
        xw_ref[j] = dinv * xt_ref[j]
    xw_ref[3] = dinv


def _tc_mid(s0_ref, s1_ref, xt_ref, dinv_ref, inv_ref, w1_ref, b1_ref,
            w2_ref, b2_ref, u_ref, g_ref, t_ref):
    dinv = dinv_ref[...]
    inv = inv_ref[...]
    sc = [s0_ref[j] + s1_ref[j] for j in range(4)]
    t_ref[...] = sc[3]
    g = [None, None]
    for cc in range(2):
        acc = None
        for k in range(16):
            hk = (sc[0] * w1_ref[k, 0] + sc[1] * w1_ref[k, 1]
                  + sc[2] * w1_ref[k, 2] + sc[3] * b1_ref[k]) * dinv
            hk = hk + (xt_ref[0] * w1_ref[k, 0] + xt_ref[1] * w1_ref[k, 1]
                       + xt_ref[2] * w1_ref[k, 2] + b1_ref[k]) * inv
            rk = jnp.maximum(hk, 0.0)
            term = rk * w2_ref[cc, k]
            acc = term if acc is None else acc + term
        g[cc] = acc
    for cc in range(2):
        g_ref[cc] = g[cc]
        u_ref[cc] = dinv * g[cc]


def _tc_fin(v0_ref, v1_ref, t_ref, dinv_ref, inv_ref, g_ref, b2_ref, o_ref):
    dinv = dinv_ref[...]
    inv = inv_ref[...]
    t = t_ref[...]
    h = [None, None]
    for cc in range(2):
        v = v0_ref[cc] + v1_ref[cc]
        h[cc] = dinv * (v + t * b2_ref[cc]) + inv * (g_ref[cc] + b2_ref[cc])
    m = jnp.maximum(h[0], h[1])
    lse = m + jnp.log(jnp.exp(h[0] - m) + jnp.exp(h[1] - m))
    for cc in range(2):
        o_ref[cc] = h[cc] - lse


def kernel(x, edge_index, edge_weight, W1, b1, W2, b2):
    n = x.shape[0]
    ne = edge_index.shape[1]
    npad = ((n + NW * LANE - 1) // (NW * LANE)) * (NW * LANE)
    nrows = npad // LANE
    nchunks = (ne + NW * CH - 1) // (NW * CH)
    nepad = NW * nchunks * CH
    pad = nepad - ne

    row = edge_index[0].astype(jnp.int32)
    col = edge_index[1].astype(jnp.int32)
    ew = edge_weight.astype(jnp.float32)
    if pad:
        spread = n + (jnp.arange(pad, dtype=jnp.int32) % (npad - n))
        row = jnp.concatenate([row, spread])
        col = jnp.concatenate([col, spread])
        ew = jnp.concatenate([ew, jnp.zeros((pad,), jnp.float32)])
    row2d = row.reshape(nepad // LANE, LANE)
    col2d = col.reshape(nepad // LANE, LANE)

    xp = jnp.pad(x.astype(jnp.float32), ((0, npad - n), (0, 0)))
    xt = xp.T.reshape(3, nrows, LANE)
    zeros1 = jnp.zeros((npad,), jnp.float32)
    zeros4 = jnp.zeros((npad * 4,), jnp.float32)
    zeros2 = jnp.zeros((npad * 2,), jnp.float32)

    deg_k, agg4_k, agg2_k = _make_sc_kernels(npad, nchunks)

    degp = deg_k(col2d, zeros1).reshape(NC, nrows, LANE)

    smem = pl.BlockSpec(memory_space=pltpu.SMEM)
    xw_t, dinv, inv = pl.pallas_call(
        _tc_prep,
        out_shape=[jax.ShapeDtypeStruct((4, nrows, LANE), jnp.float32),
                   jax.ShapeDtypeStruct((nrows, LANE), jnp.float32),
                   jax.ShapeDtypeStruct((nrows, LANE), jnp.float32)],
    )(degp, xt)
    xw = xw_t.reshape(4, npad).T.reshape(npad * 4)

    sp = agg4_k(row2d, col2d, ew, xw, zeros4)
    sp_t = sp.reshape(NC, npad, 4).transpose(0, 2, 1).reshape(NC, 4, nrows, LANE)

    u_t, g_t, tcol = pl.pallas_call(
        _tc_mid,
        in_specs=[pl.BlockSpec(), pl.BlockSpec(), pl.BlockSpec(),
                  pl.BlockSpec(), pl.BlockSpec(), smem, smem, smem, smem],
        out_shape=[jax.ShapeDtypeStruct((2, nrows, LANE), jnp.float32),
                   jax.ShapeDtypeStruct((2, nrows, LANE), jnp.float32),
                   jax.ShapeDtypeStruct((nrows, LANE), jnp.float32)],
    )(sp_t[0], sp_t[1], xt, dinv, inv, W1, b1, W2, b2)
    u = u_t.reshape(2, npad).T.reshape(npad * 2)

    vp = agg2_k(row2d, col2d, ew, u, zeros2)
    vp_t = vp.reshape(NC, npad, 2).transpose(0, 2, 1).reshape(NC, 2, nrows, LANE)

    o_t = pl.pallas_call(
        _tc_fin,
        in_specs=[pl.BlockSpec(), pl.BlockSpec(), pl.BlockSpec(),
                  pl.BlockSpec(), pl.BlockSpec(), pl.BlockSpec(), smem],
        out_shape=jax.ShapeDtypeStruct((2, nrows, LANE), jnp.float32),
    )(vp_t[0], vp_t[1], tcol, dinv, inv, g_t, b2)
    return o_t.reshape(2, npad).T[:n]
